# baseline (speedup 1.0000x reference)
import jax
import jax.numpy as jnp
from jax.experimental import pallas as pl

_K = 10
_N = 2048
_RB = 2048
_NLANE = 128
_NSLICE = _N // _NLANE
_IDX_BITS = 11
_IDX_MASK = (1 << _IDX_BITS) - 1
_IBIG = 0x7F000000


def _knn_avg_kernel(pct_ref, rows_ref, out_ref):
    r = pl.program_id(1)
    q3 = pct_ref[0]
    p = rows_ref[0]
    sq_p = jnp.sum(p * p, axis=1, keepdims=True)
    sq_q = jnp.sum(q3 * q3, axis=0, keepdims=True)
    dot = jax.lax.dot_general(p, q3, (((1,), (0,)), ((), ())),
                              preferred_element_type=jnp.float32)
    d2 = jnp.maximum(sq_p + sq_q - 2.0 * dot, 1e-12)
    ikeys = jax.lax.bitcast_convert_type(d2, jnp.int32)
    col = jax.lax.broadcasted_iota(jnp.int32, (_RB, _N), 1)
    ikeys = (ikeys & ~_IDX_MASK) | col
    row = jax.lax.broadcasted_iota(jnp.int32, (_RB, _N), 0) + r * _RB
    ikeys = jnp.where(col == row, _IBIG, ikeys)
    keys = jax.lax.bitcast_convert_type(ikeys, jnp.float32)
    fbig = jax.lax.bitcast_convert_type(
        jnp.full((1, 1), _IBIG, jnp.int32), jnp.float32)
    big = fbig[0, 0]
    big4 = jnp.full((_RB, _NLANE), big, jnp.float32)
    laneiota = jax.lax.broadcasted_iota(jnp.int32, (_RB, _NLANE), 1)

    l1 = l2 = l3 = l4 = big4
    for s in range(_NSLICE):
        x = keys[:, s * _NLANE:(s + 1) * _NLANE]
        t1 = jnp.maximum(l1, x)
        l1 = jnp.minimum(l1, x)
        t2 = jnp.maximum(l2, t1)
        l2 = jnp.minimum(l2, t1)
        t3 = jnp.maximum(l3, t2)
        l3 = jnp.minimum(l3, t2)
        l4 = jnp.minimum(l4, t3)

    def extract10(cands, width_iota):
        acc = jnp.zeros((_RB, 1), jnp.float32)
        hits = jnp.zeros((_RB, _NLANE), jnp.int32)
        mprev = None
        for t in range(_K):
            m = None
            for c in cands:
                cm = c if mprev is None else jnp.where(c > mprev, c, big)
                part = jnp.min(cm, axis=1, keepdims=True)
                m = part if m is None else jnp.minimum(m, part)
            mi = jax.lax.bitcast_convert_type(m, jnp.int32)
            val = jax.lax.bitcast_convert_type(mi & ~_IDX_MASK, jnp.float32)
            acc = acc + jnp.sqrt(val)
            if width_iota is not None:
                hit = width_iota == ((mi & _IDX_MASK) & (_NLANE - 1))
                hits = hits + hit.astype(jnp.int32)
            mprev = m
        return acc, hits

    mvec = big4
    hits = jnp.zeros((_RB, _NLANE), jnp.int32)
    mb = None
    for t in range(_K):
        if mb is None:
            nxt = l1
        else:
            nxt = jnp.where(l1 > mb, l1,
                            jnp.where(l2 > mb, l2,
                                      jnp.where(l3 > mb, l3,
                                                jnp.where(l4 > mb, l4,
                                                          big))))
        m = jnp.min(nxt, axis=1, keepdims=True)
        mb = jnp.broadcast_to(m, (_RB, _NLANE))
        hit = nxt == mb
        hits = hits + hit.astype(jnp.int32)
        mvec = jnp.where(laneiota == t, mb, mvec)
    vi = jax.lax.bitcast_convert_type(mvec, jnp.int32)
    vals = jax.lax.bitcast_convert_type(vi & ~_IDX_MASK, jnp.float32)
    roots = jnp.where(laneiota < _K, jnp.sqrt(vals), 0.0)
    acc = jnp.sum(roots, axis=1, keepdims=True)

    exh_row = jnp.max(hits, axis=1, keepdims=True) >= 4

    def fix():
        slow_acc, _ = extract10(
            tuple(keys[:, s * _NLANE:(s + 1) * _NLANE]
                  for s in range(_NSLICE)), None)
        return jnp.where(exh_row, slow_acc, acc)

    acc = jax.lax.cond(jnp.any(exh_row), fix, lambda: acc)
    out_ref[0, :, :] = acc * (1.0 / _K)


def kernel(point_cloud):
    B, N, D = point_cloud.shape
    pct = jnp.transpose(point_cloud, (0, 2, 1))
    nrb = N // _RB
    out = pl.pallas_call(
        _knn_avg_kernel,
        grid=(B, nrb),
        in_specs=[
            pl.BlockSpec((1, D, N), lambda b, r: (b, 0, 0)),
            pl.BlockSpec((1, _RB, D), lambda b, r: (b, r, 0)),
        ],
        out_specs=pl.BlockSpec((1, _RB, 1), lambda b, r: (b * nrb + r, 0, 0)),
        out_shape=jax.ShapeDtypeStruct((B * nrb, _RB, 1), jnp.float32),
    )(pct, point_cloud)
    avg = out.reshape(B, N)
    var = jnp.var(avg, axis=-1, ddof=1)
    return jnp.mean(var)


# speedup vs baseline: 2.7113x; 2.7113x over previous
import jax
import jax.numpy as jnp
from jax.experimental import pallas as pl

_K = 10
_N = 2048
_RB = 1024
_NLANE = 128
_NSLICE = _N // _NLANE
_IDX_BITS = 11
_IDX_MASK = (1 << _IDX_BITS) - 1
_IBIG = 0x7F000000


def _knn_avg_kernel(pct_ref, rows_ref, out_ref):
    r = pl.program_id(1)
    q3 = pct_ref[0]
    p = rows_ref[0]
    sq_p = jnp.sum(p * p, axis=1, keepdims=True)
    sq_q = jnp.sum(q3 * q3, axis=0, keepdims=True)
    p_aug = jnp.concatenate(
        [p * -2.0, sq_p, jnp.ones((_RB, 1), jnp.float32)], axis=1)
    q_aug = jnp.concatenate(
        [q3, jnp.ones((1, _N), jnp.float32), sq_q], axis=0)
    raw = jax.lax.dot_general(p_aug, q_aug, (((1,), (0,)), ((), ())),
                              preferred_element_type=jnp.float32)
    d2 = jnp.maximum(raw, 1e-12)
    ikeys = jax.lax.bitcast_convert_type(d2, jnp.int32)
    col = jax.lax.broadcasted_iota(jnp.int32, (_RB, _N), 1)
    ikeys = (ikeys & ~_IDX_MASK) | col
    row = jax.lax.broadcasted_iota(jnp.int32, (_RB, _N), 0) + r * _RB
    ikeys = jnp.where(col == row, _IBIG, ikeys)
    keys = jax.lax.bitcast_convert_type(ikeys, jnp.float32)
    fbig = jax.lax.bitcast_convert_type(
        jnp.full((1, 1), _IBIG, jnp.int32), jnp.float32)
    big = fbig[0, 0]
    big4 = jnp.full((_RB, _NLANE), big, jnp.float32)
    laneiota = jax.lax.broadcasted_iota(jnp.int32, (_RB, _NLANE), 1)

    l1 = l2 = l3 = l4 = big4
    for s in range(_NSLICE):
        x = keys[:, s * _NLANE:(s + 1) * _NLANE]
        t1 = jnp.maximum(l1, x)
        l1 = jnp.minimum(l1, x)
        t2 = jnp.maximum(l2, t1)
        l2 = jnp.minimum(l2, t1)
        t3 = jnp.maximum(l3, t2)
        l3 = jnp.minimum(l3, t2)
        l4 = jnp.minimum(l4, t3)

    def extract10(cands, width_iota):
        acc = jnp.zeros((_RB, 1), jnp.float32)
        hits = jnp.zeros((_RB, _NLANE), jnp.int32)
        mprev = None
        for t in range(_K):
            m = None
            for c in cands:
                cm = c if mprev is None else jnp.where(c > mprev, c, big)
                part = jnp.min(cm, axis=1, keepdims=True)
                m = part if m is None else jnp.minimum(m, part)
            mi = jax.lax.bitcast_convert_type(m, jnp.int32)
            val = jax.lax.bitcast_convert_type(mi & ~_IDX_MASK, jnp.float32)
            acc = acc + jnp.sqrt(val)
            if width_iota is not None:
                hit = width_iota == ((mi & _IDX_MASK) & (_NLANE - 1))
                hits = hits + hit.astype(jnp.int32)
            mprev = m
        return acc, hits

    mvec = big4
    mb = None
    for t in range(_K):
        if mb is None:
            nxt = l1
        else:
            nxt = jnp.where(l1 > mb, l1,
                            jnp.where(l2 > mb, l2,
                                      jnp.where(l3 > mb, l3,
                                                jnp.where(l4 > mb, l4,
                                                          big))))
        m = jnp.min(nxt, axis=1, keepdims=True)
        mb = jnp.broadcast_to(m, (_RB, _NLANE))
        mvec = jnp.where(laneiota == t, mb, mvec)
    vi = jax.lax.bitcast_convert_type(mvec, jnp.int32)
    vals = jax.lax.bitcast_convert_type(vi & ~_IDX_MASK, jnp.float32)
    roots = jnp.where(laneiota < _K, jnp.sqrt(vals), 0.0)
    acc = jnp.sum(roots, axis=1, keepdims=True)

    exh_row = jnp.any(l4 <= mb, axis=1, keepdims=True)

    def fix():
        slow_acc, _ = extract10(
            tuple(keys[:, s * _NLANE:(s + 1) * _NLANE]
                  for s in range(_NSLICE)), None)
        return jnp.where(exh_row, slow_acc, acc)

    acc = jax.lax.cond(jnp.any(exh_row), fix, lambda: acc)
    out_ref[0, :, :] = acc * (1.0 / _K)


def kernel(point_cloud):
    B, N, D = point_cloud.shape
    pct = jnp.transpose(point_cloud, (0, 2, 1))
    nrb = N // _RB
    out = pl.pallas_call(
        _knn_avg_kernel,
        grid=(B, nrb),
        in_specs=[
            pl.BlockSpec((1, D, N), lambda b, r: (b, 0, 0)),
            pl.BlockSpec((1, _RB, D), lambda b, r: (b, r, 0)),
        ],
        out_specs=pl.BlockSpec((1, _RB, 1), lambda b, r: (b * nrb + r, 0, 0)),
        out_shape=jax.ShapeDtypeStruct((B * nrb, _RB, 1), jnp.float32),
    )(pct, point_cloud)
    avg = out.reshape(B, N)
    var = jnp.var(avg, axis=-1, ddof=1)
    return jnp.mean(var)
